# baseline (speedup 1.0000x reference)
import functools

import jax
import jax.numpy as jnp
from jax import lax
from jax.experimental import pallas as pl
from jax.experimental.pallas import tpu as pltpu
from jax.experimental.pallas import tpu_sc as plsc

DIM = 32
NC = 2
NS = 16
NW = NC * NS
LB = 128


@functools.lru_cache(maxsize=None)
def _make_gather(batch: int, width: int, vocab: int):
    nb = batch // LB
    units = width * nb
    assert batch % LB == 0 and units % (2 * NW) == 0, (batch, width)
    upw = units // NW
    db = DIM // 8
    mesh = plsc.VectorSubcoreMesh(
        core_axis_name="c", subcore_axis_name="s",
        num_cores=NC, num_subcores=NS,
    )

    @functools.partial(
        pl.kernel,
        out_type=jax.ShapeDtypeStruct((width, db, nb, 8, LB), jnp.float32),
        mesh=mesh,
        scratch_types=[
            pltpu.VMEM((upw * LB,), jnp.int32),
            pltpu.VMEM((2, LB, DIM), jnp.float32),
            pltpu.VMEM((2, DIM, LB), jnp.float32),
            pltpu.SemaphoreType.DMA,
            pltpu.SemaphoreType.DMA,
            pltpu.SemaphoreType.DMA,
            pltpu.SemaphoreType.DMA,
        ],
        compiler_params=pltpu.CompilerParams(
            use_tc_tiling_on_sc=False, needs_layout_passes=False),
    )
    def k(idx_hbm, table_hbm, om_hbm, idx_v, rows_v, ot_v,
          gsem0, gsem1, wsem0, wsem1):
        gsems = (gsem0, gsem1)
        wsems = (wsem0, wsem1)
        wid = lax.axis_index("s") * NC + lax.axis_index("c")
        ubase = wid * upw
        pltpu.sync_copy(idx_hbm.at[pl.ds(ubase * LB, upw * LB)], idx_v)

        iota16 = lax.iota(jnp.int32, 16)

        def fire(ul, slot):
            pltpu.async_copy(
                table_hbm.at[idx_v.at[pl.ds(ul * LB, LB)]],
                rows_v.at[slot], gsems[slot])

        def drain(ul, slot):
            pltpu.make_async_copy(
                table_hbm.at[idx_v.at[pl.ds(ul * LB, LB)]],
                rows_v.at[slot], gsems[slot]).wait()

        def wb(w, bb, slot):
            for d0 in range(db):
                pltpu.async_copy(
                    ot_v.at[slot, pl.ds(d0 * 8, 8)],
                    om_hbm.at[w, d0, bb], wsems[slot])

        def wb_wait(w, bb, slot):
            for d0 in range(db):
                pltpu.make_async_copy(
                    ot_v.at[slot, pl.ds(d0 * 8, 8)],
                    om_hbm.at[w, d0, bb], wsems[slot]).wait()

        def transpose(slot):
            rows = rows_v.at[slot]
            for d in range(DIM):
                cvec = jnp.full((16,), d, jnp.int32)
                for l0 in range(0, LB, 16):
                    v = plsc.load_gather(rows, [iota16 + l0, cvec])
                    ot_v[slot, d, pl.ds(l0, 16)] = v

        fire(0, 0)

        @pl.loop(0, upw, step=2)
        def _pair(u0):
            for b in range(2):
                ul = u0 + b
                u = ubase + ul
                w = lax.div(u, nb)
                bb = lax.rem(u, nb)

                @pl.when(ul + 1 < upw)
                def _():
                    fire(ul + 1, 1 - b)

                drain(ul, b)

                @pl.when(ul >= 2)
                def _():
                    wb_wait(w, bb, b)

                transpose(b)
                wb(w, bb, b)

        wb_wait(0, 0, 0)
        wb_wait(0, 0, 1)

    return k


def kernel(input, table):
    batch, width = input.shape
    vocab, dim = table.shape
    assert dim == DIM
    idx_wm = input.T.reshape(batch * width).astype(jnp.int32)
    om = _make_gather(batch, width, vocab)(idx_wm, table)
    out = jnp.transpose(om, (2, 4, 0, 1, 3)).reshape(batch, width, dim)
    return out
